# baseline (speedup 1.0000x reference)
import functools

import jax
import jax.numpy as jnp
from jax import lax
from jax.experimental import pallas as pl
from jax.experimental.pallas import tpu as pltpu
from jax.experimental.pallas import tpu_sc as plsc

NCHUNK = 4


def _sc_gather(table, idx, chunk=64):
    n_tokens = idx.shape[0]
    d = table.shape[1]
    info = plsc.get_sparse_core_info()
    num_workers = info.num_cores * info.num_subcores
    per_worker = n_tokens // num_workers
    n_chunks = per_worker // chunk
    mesh = plsc.VectorSubcoreMesh(core_axis_name="c", subcore_axis_name="s")

    @functools.partial(
        pl.kernel,
        mesh=mesh,
        out_type=jax.ShapeDtypeStruct((n_tokens, d), jnp.float32),
        scratch_types=[
            pltpu.VMEM((chunk,), jnp.int32),
            pltpu.VMEM((chunk, d), jnp.float32),
            pltpu.SemaphoreType.DMA,
        ],
    )
    def gather_kernel(table_hbm, idx_hbm, out_hbm, idx_v, rows_v, sem):
        wid = lax.axis_index("s") * info.num_cores + lax.axis_index("c")
        base = wid * per_worker

        def body(i, carry):
            off = base + i * chunk
            pltpu.sync_copy(idx_hbm.at[pl.ds(off, chunk)], idx_v)
            pltpu.async_copy(table_hbm.at[idx_v], rows_v, sem).wait()
            pltpu.sync_copy(rows_v, out_hbm.at[pl.ds(off, chunk)])
            return carry

        lax.fori_loop(0, n_chunks, body, 0)

    return gather_kernel(table, idx)


def _tc_body(g_ref, sv_ref, sw_ref, pw_ref, o_ref, *, scale):
    e = g_ref[...] + lax.dot_general(
        sv_ref[...], sw_ref[...],
        (((1,), (0,)), ((), ())),
        preferred_element_type=jnp.float32,
    )
    o_ref[...] = lax.dot_general(
        e.astype(jnp.bfloat16), pw_ref[...].astype(jnp.bfloat16),
        (((1,), (1,)), ((), ())),
        preferred_element_type=jnp.float32,
    ) * scale


def _tc_project_chunk(prev, g, sv, sw, pw, n_total, block_off, block_t=256):
    ct, d_embed = g.shape
    d_proj = pw.shape[0]
    vec_len = sv.shape[1]
    scale = float(d_proj) ** 0.5
    body = functools.partial(_tc_body, scale=scale)

    def chunk_body(prev_ref, g_ref, sv_ref, sw_ref, pw_ref, o_ref):
        del prev_ref
        body(g_ref, sv_ref, sw_ref, pw_ref, o_ref)

    return pl.pallas_call(
        chunk_body,
        grid=(ct // block_t,),
        in_specs=[
            pl.BlockSpec(memory_space=pl.ANY),
            pl.BlockSpec((block_t, d_embed), lambda i: (i, 0)),
            pl.BlockSpec((block_t, vec_len), lambda i: (i, 0)),
            pl.BlockSpec((vec_len, d_embed), lambda i: (0, 0)),
            pl.BlockSpec((d_proj, d_embed), lambda i: (0, 0)),
        ],
        out_specs=pl.BlockSpec(
            (block_t, d_proj), lambda i, _o=block_off: (i + _o, 0)),
        out_shape=jax.ShapeDtypeStruct((n_total, d_proj), jnp.float32),
        input_output_aliases={0: 0},
    )(prev, g, sv, sw, pw)


def _tc_project_first(g, sv, sw, pw, n_total, block_t=256):
    ct, d_embed = g.shape
    d_proj = pw.shape[0]
    vec_len = sv.shape[1]
    scale = float(d_proj) ** 0.5
    body = functools.partial(_tc_body, scale=scale)

    return pl.pallas_call(
        body,
        grid=(ct // block_t,),
        in_specs=[
            pl.BlockSpec((block_t, d_embed), lambda i: (i, 0)),
            pl.BlockSpec((block_t, vec_len), lambda i: (i, 0)),
            pl.BlockSpec((vec_len, d_embed), lambda i: (0, 0)),
            pl.BlockSpec((d_proj, d_embed), lambda i: (0, 0)),
        ],
        out_specs=pl.BlockSpec((block_t, d_proj), lambda i: (i, 0)),
        out_shape=jax.ShapeDtypeStruct((n_total, d_proj), jnp.float32),
    )(g, sv, sw, pw)


def kernel(inp, status_vec, emb_weight, status_weight, proj_W):
    b, l = inp.shape
    n_tokens = b * l
    d_proj = proj_W.shape[0]
    block_t = 256
    ct = n_tokens // NCHUNK
    idx = inp.reshape(n_tokens).astype(jnp.int32)
    sv = status_vec.reshape(n_tokens, status_vec.shape[-1])

    gs = [_sc_gather(emb_weight, idx[k * ct:(k + 1) * ct])
          for k in range(NCHUNK)]
    out = _tc_project_first(gs[0], sv[:ct], status_weight, proj_W,
                            n_tokens, block_t)
    for k in range(1, NCHUNK):
        out = _tc_project_chunk(
            out, gs[k], sv[k * ct:(k + 1) * ct], status_weight, proj_W,
            n_tokens, block_off=k * (ct // block_t), block_t=block_t)
    return out.reshape(b, l, d_proj)


# speedup vs baseline: 1.1546x; 1.1546x over previous
import functools

import jax
import jax.numpy as jnp
from jax import lax
from jax.experimental import pallas as pl
from jax.experimental.pallas import tpu as pltpu
from jax.experimental.pallas import tpu_sc as plsc

NCHUNK = 1


def _sc_gather(table, idx, chunk=64):
    n_tokens = idx.shape[0]
    d = table.shape[1]
    info = plsc.get_sparse_core_info()
    num_workers = info.num_cores * info.num_subcores
    per_worker = n_tokens // num_workers
    n_chunks = per_worker // chunk
    mesh = plsc.VectorSubcoreMesh(core_axis_name="c", subcore_axis_name="s")

    @functools.partial(
        pl.kernel,
        mesh=mesh,
        out_type=jax.ShapeDtypeStruct((n_tokens, d), jnp.float32),
        scratch_types=[
            pltpu.VMEM((chunk,), jnp.int32),
            pltpu.VMEM((chunk, d), jnp.float32),
            pltpu.SemaphoreType.DMA,
        ],
    )
    def gather_kernel(table_hbm, idx_hbm, out_hbm, idx_v, rows_v, sem):
        wid = lax.axis_index("s") * info.num_cores + lax.axis_index("c")
        base = wid * per_worker

        def body(i, carry):
            off = base + i * chunk
            pltpu.sync_copy(idx_hbm.at[pl.ds(off, chunk)], idx_v)
            pltpu.async_copy(table_hbm.at[idx_v], rows_v, sem).wait()
            pltpu.sync_copy(rows_v, out_hbm.at[pl.ds(off, chunk)])
            return carry

        lax.fori_loop(0, n_chunks, body, 0)

    return gather_kernel(table, idx)


def _tc_body(g_ref, sv_ref, sw_ref, pw_ref, o_ref, *, scale):
    e = g_ref[...] + lax.dot_general(
        sv_ref[...], sw_ref[...],
        (((1,), (0,)), ((), ())),
        preferred_element_type=jnp.float32,
    )
    o_ref[...] = lax.dot_general(
        e.astype(jnp.bfloat16), pw_ref[...].astype(jnp.bfloat16),
        (((1,), (1,)), ((), ())),
        preferred_element_type=jnp.float32,
    ) * scale


def _tc_project_chunk(prev, g, sv, sw, pw, n_total, block_off, block_t=256):
    ct, d_embed = g.shape
    d_proj = pw.shape[0]
    vec_len = sv.shape[1]
    scale = float(d_proj) ** 0.5
    body = functools.partial(_tc_body, scale=scale)

    def chunk_body(prev_ref, g_ref, sv_ref, sw_ref, pw_ref, o_ref):
        del prev_ref
        body(g_ref, sv_ref, sw_ref, pw_ref, o_ref)

    return pl.pallas_call(
        chunk_body,
        grid=(ct // block_t,),
        in_specs=[
            pl.BlockSpec(memory_space=pl.ANY),
            pl.BlockSpec((block_t, d_embed), lambda i: (i, 0)),
            pl.BlockSpec((block_t, vec_len), lambda i: (i, 0)),
            pl.BlockSpec((vec_len, d_embed), lambda i: (0, 0)),
            pl.BlockSpec((d_proj, d_embed), lambda i: (0, 0)),
        ],
        out_specs=pl.BlockSpec(
            (block_t, d_proj), lambda i, _o=block_off: (i + _o, 0)),
        out_shape=jax.ShapeDtypeStruct((n_total, d_proj), jnp.float32),
        input_output_aliases={0: 0},
    )(prev, g, sv, sw, pw)


def _tc_project_first(g, sv, sw, pw, n_total, block_t=256):
    ct, d_embed = g.shape
    d_proj = pw.shape[0]
    vec_len = sv.shape[1]
    scale = float(d_proj) ** 0.5
    body = functools.partial(_tc_body, scale=scale)

    return pl.pallas_call(
        body,
        grid=(ct // block_t,),
        in_specs=[
            pl.BlockSpec((block_t, d_embed), lambda i: (i, 0)),
            pl.BlockSpec((block_t, vec_len), lambda i: (i, 0)),
            pl.BlockSpec((vec_len, d_embed), lambda i: (0, 0)),
            pl.BlockSpec((d_proj, d_embed), lambda i: (0, 0)),
        ],
        out_specs=pl.BlockSpec((block_t, d_proj), lambda i: (i, 0)),
        out_shape=jax.ShapeDtypeStruct((n_total, d_proj), jnp.float32),
    )(g, sv, sw, pw)


def kernel(inp, status_vec, emb_weight, status_weight, proj_W):
    b, l = inp.shape
    n_tokens = b * l
    d_proj = proj_W.shape[0]
    block_t = 512
    ct = n_tokens // NCHUNK
    idx = inp.reshape(n_tokens).astype(jnp.int32)
    sv = status_vec.reshape(n_tokens, status_vec.shape[-1])

    gs = [_sc_gather(emb_weight, idx[k * ct:(k + 1) * ct])
          for k in range(NCHUNK)]
    out = _tc_project_first(gs[0], sv[:ct], status_weight, proj_W,
                            n_tokens, block_t)
    for k in range(1, NCHUNK):
        out = _tc_project_chunk(
            out, gs[k], sv[k * ct:(k + 1) * ct], status_weight, proj_W,
            n_tokens, block_off=k * (ct // block_t), block_t=block_t)
    return out.reshape(b, l, d_proj)


# speedup vs baseline: 1.1807x; 1.0227x over previous
import functools

import jax
import jax.numpy as jnp
from jax import lax
from jax.experimental import pallas as pl
from jax.experimental.pallas import tpu as pltpu
from jax.experimental.pallas import tpu_sc as plsc

NCHUNK = 1


def _sc_gather(table, idx, chunk=64):
    n_tokens = idx.shape[0]
    d = table.shape[1]
    info = plsc.get_sparse_core_info()
    num_workers = info.num_cores * info.num_subcores
    per_worker = n_tokens // num_workers
    n_chunks = per_worker // chunk
    mesh = plsc.VectorSubcoreMesh(core_axis_name="c", subcore_axis_name="s")

    @functools.partial(
        pl.kernel,
        mesh=mesh,
        out_type=jax.ShapeDtypeStruct((n_tokens, d), jnp.float32),
        scratch_types=[
            pltpu.VMEM((chunk,), jnp.int32),
            pltpu.VMEM((chunk, d), jnp.float32),
            pltpu.SemaphoreType.DMA,
        ],
    )
    def gather_kernel(table_hbm, idx_hbm, out_hbm, idx_v, rows_v, sem):
        wid = lax.axis_index("s") * info.num_cores + lax.axis_index("c")
        base = wid * per_worker

        def body(i, carry):
            off = base + i * chunk
            pltpu.sync_copy(idx_hbm.at[pl.ds(off, chunk)], idx_v)
            pltpu.async_copy(table_hbm.at[idx_v], rows_v, sem).wait()
            pltpu.sync_copy(rows_v, out_hbm.at[pl.ds(off, chunk)])
            return carry

        lax.fori_loop(0, n_chunks, body, 0)

    return gather_kernel(table, idx)


def _tc_body(g_ref, sv_ref, sw_ref, pw_ref, o_ref, *, scale):
    e = g_ref[...] + lax.dot_general(
        sv_ref[...], sw_ref[...],
        (((1,), (0,)), ((), ())),
        preferred_element_type=jnp.float32,
    )
    o_ref[...] = lax.dot_general(
        e.astype(jnp.bfloat16), pw_ref[...].astype(jnp.bfloat16),
        (((1,), (1,)), ((), ())),
        preferred_element_type=jnp.float32,
    ) * scale


def _tc_project_chunk(prev, g, sv, sw, pw, n_total, block_off, block_t=256):
    ct, d_embed = g.shape
    d_proj = pw.shape[0]
    vec_len = sv.shape[1]
    scale = float(d_proj) ** 0.5
    body = functools.partial(_tc_body, scale=scale)

    def chunk_body(prev_ref, g_ref, sv_ref, sw_ref, pw_ref, o_ref):
        del prev_ref
        body(g_ref, sv_ref, sw_ref, pw_ref, o_ref)

    return pl.pallas_call(
        chunk_body,
        grid=(ct // block_t,),
        in_specs=[
            pl.BlockSpec(memory_space=pl.ANY),
            pl.BlockSpec((block_t, d_embed), lambda i: (i, 0)),
            pl.BlockSpec((block_t, vec_len), lambda i: (i, 0)),
            pl.BlockSpec((vec_len, d_embed), lambda i: (0, 0)),
            pl.BlockSpec((d_proj, d_embed), lambda i: (0, 0)),
        ],
        out_specs=pl.BlockSpec(
            (block_t, d_proj), lambda i, _o=block_off: (i + _o, 0)),
        out_shape=jax.ShapeDtypeStruct((n_total, d_proj), jnp.float32),
        input_output_aliases={0: 0},
    )(prev, g, sv, sw, pw)


def _tc_project_first(g, sv, sw, pw, n_total, block_t=256):
    ct, d_embed = g.shape
    d_proj = pw.shape[0]
    vec_len = sv.shape[1]
    scale = float(d_proj) ** 0.5
    body = functools.partial(_tc_body, scale=scale)

    return pl.pallas_call(
        body,
        grid=(ct // block_t,),
        in_specs=[
            pl.BlockSpec((block_t, d_embed), lambda i: (i, 0)),
            pl.BlockSpec((block_t, vec_len), lambda i: (i, 0)),
            pl.BlockSpec((vec_len, d_embed), lambda i: (0, 0)),
            pl.BlockSpec((d_proj, d_embed), lambda i: (0, 0)),
        ],
        out_specs=pl.BlockSpec((block_t, d_proj), lambda i: (i, 0)),
        out_shape=jax.ShapeDtypeStruct((n_total, d_proj), jnp.float32),
    )(g, sv, sw, pw)


def kernel(inp, status_vec, emb_weight, status_weight, proj_W):
    b, l = inp.shape
    n_tokens = b * l
    d_proj = proj_W.shape[0]
    block_t = 1024
    ct = n_tokens // NCHUNK
    idx = inp.reshape(n_tokens).astype(jnp.int32)
    sv = status_vec.reshape(n_tokens, status_vec.shape[-1])

    gs = [_sc_gather(emb_weight, idx[k * ct:(k + 1) * ct])
          for k in range(NCHUNK)]
    out = _tc_project_first(gs[0], sv[:ct], status_weight, proj_W,
                            n_tokens, block_t)
    for k in range(1, NCHUNK):
        out = _tc_project_chunk(
            out, gs[k], sv[k * ct:(k + 1) * ct], status_weight, proj_W,
            n_tokens, block_off=k * (ct // block_t), block_t=block_t)
    return out.reshape(b, l, d_proj)
